# Initial kernel scaffold; baseline (speedup 1.0000x reference)
import jax
import jax.numpy as jnp
from jax.experimental import pallas as pl


def kernel(x, edge_index, edge_label_index, W_l1, b_l1, W_r1, W_l2, b_l2, W_r2):
    raise NotImplementedError("write your pallas kernel here")


# speedup vs baseline: 3.4275x; 3.4275x over previous
import jax
import jax.numpy as jnp
from jax import lax
from jax.experimental import pallas as pl
from jax.experimental.pallas import tpu as pltpu
from jax.experimental.pallas import tpu_sc as plsc

NC = 2
NS = 16
NW = NC * NS
LANES = 16

NPAD = 10240
STRIPE = NPAD // NS
ECH = 128
ENCH = 80
EPW = ENCH * ECH
EP = NW * EPW
DCH = 128
DNCH = 25
DPW = DNCH * DCH
ELP = NW * DPW


def _mesh():
    return plsc.VectorSubcoreMesh(core_axis_name="c", subcore_axis_name="s")


def _tc1_body(x_ref, wl_ref, wr_ref, b_ref, y_ref, r_ref):
    xv = x_ref[...]
    y_ref[...] = jnp.dot(xv, wl_ref[...], preferred_element_type=jnp.float32)
    r_ref[...] = (
        jnp.dot(xv, wr_ref[...], preferred_element_type=jnp.float32) + b_ref[...]
    )


def _tc1(x, W_l, W_r, b, br=2000):
    n, d = x.shape
    dh = W_l.shape[1]
    grid = n // br
    return pl.pallas_call(
        _tc1_body,
        grid=(grid,),
        in_specs=[
            pl.BlockSpec((br, d), lambda i: (i, 0)),
            pl.BlockSpec((d, dh), lambda i: (0, 0)),
            pl.BlockSpec((d, dh), lambda i: (0, 0)),
            pl.BlockSpec((1, dh), lambda i: (0, 0)),
        ],
        out_specs=[
            pl.BlockSpec((br, dh), lambda i: (i, 0)),
            pl.BlockSpec((br, dh), lambda i: (i, 0)),
        ],
        out_shape=[
            jax.ShapeDtypeStruct((n, dh), jnp.float32),
            jax.ShapeDtypeStruct((n, dh), jnp.float32),
        ],
    )(x, W_l, W_r, b.reshape(1, dh))


def _tc2_body(aggA, aggB, dgA, dgB, r1_ref, wl_ref, wr_ref, b_ref, y2_ref, r2_ref):
    deg = dgA[...] + dgB[...]
    inv = 1.0 / jnp.maximum(deg, 1.0)
    h = jnp.maximum((aggA[...] + aggB[...]) * inv + r1_ref[...], 0.0)
    y2_ref[...] = jnp.dot(h, wl_ref[...], preferred_element_type=jnp.float32)
    r2_ref[...] = (
        jnp.dot(h, wr_ref[...], preferred_element_type=jnp.float32) + b_ref[...]
    )


def _tc2(agg, deg2d, r1, W_l2, W_r2, b_l2, br=2048):
    dh = r1.shape[1]
    do = W_l2.shape[1]
    grid = NPAD // br
    off = NPAD // br
    return pl.pallas_call(
        _tc2_body,
        grid=(grid,),
        in_specs=[
            pl.BlockSpec((br, dh), lambda i: (i, 0)),
            pl.BlockSpec((br, dh), lambda i: (i + off, 0)),
            pl.BlockSpec((br, 1), lambda i: (i, 0)),
            pl.BlockSpec((br, 1), lambda i: (i + off, 0)),
            pl.BlockSpec((br, dh), lambda i: (i, 0)),
            pl.BlockSpec((dh, do), lambda i: (0, 0)),
            pl.BlockSpec((dh, do), lambda i: (0, 0)),
            pl.BlockSpec((1, do), lambda i: (0, 0)),
        ],
        out_specs=[
            pl.BlockSpec((br, do), lambda i: (i, 0)),
            pl.BlockSpec((br, do), lambda i: (i, 0)),
        ],
        out_shape=[
            jax.ShapeDtypeStruct((NPAD, do), jnp.float32),
            jax.ShapeDtypeStruct((NPAD, do), jnp.float32),
        ],
    )(agg, agg, deg2d, deg2d, r1, W_l2, W_r2, b_l2.reshape(1, do))


def _tc3_body(aggA, aggB, dgA, dgB, r2_ref, z_ref):
    deg = dgA[...] + dgB[...]
    inv = 1.0 / jnp.maximum(deg, 1.0)
    z_ref[...] = (aggA[...] + aggB[...]) * inv + r2_ref[...]


def _tc3(agg, deg2d, r2, br=2048):
    do = r2.shape[1]
    grid = NPAD // br
    off = NPAD // br
    return pl.pallas_call(
        _tc3_body,
        grid=(grid,),
        in_specs=[
            pl.BlockSpec((br, do), lambda i: (i, 0)),
            pl.BlockSpec((br, do), lambda i: (i + off, 0)),
            pl.BlockSpec((br, 1), lambda i: (i, 0)),
            pl.BlockSpec((br, 1), lambda i: (i + off, 0)),
            pl.BlockSpec((br, do), lambda i: (i, 0)),
        ],
        out_specs=pl.BlockSpec((br, do), lambda i: (i, 0)),
        out_shape=jax.ShapeDtypeStruct((NPAD, do), jnp.float32),
    )(agg, agg, deg2d, deg2d, r2)


def _sc_aggregate_deg(y, src, dst, zeros_d, zeros1, ones1):
    n, d = y.shape

    def body(y_hbm, src_hbm, dst_hbm, z_hbm, z1_hbm, ones_hbm,
             agg_out, deg_out, agg_sh, deg_sh, src_v, dst_v, rows_v,
             ones_v, sem):
        c = lax.axis_index("c")
        s = lax.axis_index("s")
        wid = c * NS + s
        row0 = s * STRIPE
        pltpu.sync_copy(z_hbm.at[pl.ds(row0, STRIPE)],
                        agg_sh.at[pl.ds(row0, STRIPE)])
        pltpu.sync_copy(z1_hbm.at[pl.ds(row0, STRIPE)],
                        deg_sh.at[pl.ds(row0, STRIPE)])
        pltpu.sync_copy(src_hbm.at[wid], src_v)
        pltpu.sync_copy(dst_hbm.at[wid], dst_v)
        pltpu.sync_copy(ones_hbm, ones_v)
        plsc.subcore_barrier()

        def chunk(j, carry):
            pltpu.async_copy(y_hbm.at[src_v.at[j]], rows_v, sem).wait()
            pltpu.sync_copy(rows_v, agg_sh.at[dst_v.at[j]], add=True)
            pltpu.sync_copy(ones_v, deg_sh.at[dst_v.at[j]], add=True)
            return carry

        lax.fori_loop(0, ENCH, chunk, 0)
        plsc.subcore_barrier()
        pltpu.sync_copy(agg_sh.at[pl.ds(row0, STRIPE)],
                        agg_out.at[pl.ds(c * NPAD + row0, STRIPE)])
        pltpu.sync_copy(deg_sh.at[pl.ds(row0, STRIPE)],
                        deg_out.at[pl.ds(c * NPAD + row0, STRIPE)])

    fn = pl.kernel(
        body,
        out_type=[
            jax.ShapeDtypeStruct((NC * NPAD, d), jnp.float32),
            jax.ShapeDtypeStruct((NC * NPAD,), jnp.float32),
        ],
        mesh=_mesh(),
        compiler_params=pltpu.CompilerParams(use_tc_tiling_on_sc=False, needs_layout_passes=False),
        scratch_types=[
            pltpu.VMEM_SHARED((NPAD, d), jnp.float32),
            pltpu.VMEM_SHARED((NPAD,), jnp.float32),
            pltpu.VMEM((ENCH, ECH), jnp.int32),
            pltpu.VMEM((ENCH, ECH), jnp.int32),
            pltpu.VMEM((ECH, d), jnp.float32),
            pltpu.VMEM((ECH,), jnp.float32),
            pltpu.SemaphoreType.DMA,
        ],
    )
    return fn(y, src, dst, zeros_d, zeros1, ones1)


def _sc_aggregate(y, src, dst, zeros_d):
    n, d = y.shape

    def body(y_hbm, src_hbm, dst_hbm, z_hbm,
             agg_out, agg_sh, src_v, dst_v, rows_v, sem):
        c = lax.axis_index("c")
        s = lax.axis_index("s")
        wid = c * NS + s
        row0 = s * STRIPE
        pltpu.sync_copy(z_hbm.at[pl.ds(row0, STRIPE)],
                        agg_sh.at[pl.ds(row0, STRIPE)])
        pltpu.sync_copy(src_hbm.at[wid], src_v)
        pltpu.sync_copy(dst_hbm.at[wid], dst_v)
        plsc.subcore_barrier()

        def chunk(j, carry):
            pltpu.async_copy(y_hbm.at[src_v.at[j]], rows_v, sem).wait()
            pltpu.sync_copy(rows_v, agg_sh.at[dst_v.at[j]], add=True)
            return carry

        lax.fori_loop(0, ENCH, chunk, 0)
        plsc.subcore_barrier()
        pltpu.sync_copy(agg_sh.at[pl.ds(row0, STRIPE)],
                        agg_out.at[pl.ds(c * NPAD + row0, STRIPE)])

    fn = pl.kernel(
        body,
        out_type=jax.ShapeDtypeStruct((NC * NPAD, d), jnp.float32),
        mesh=_mesh(),
        compiler_params=pltpu.CompilerParams(use_tc_tiling_on_sc=False, needs_layout_passes=False),
        scratch_types=[
            pltpu.VMEM_SHARED((NPAD, d), jnp.float32),
            pltpu.VMEM((ENCH, ECH), jnp.int32),
            pltpu.VMEM((ENCH, ECH), jnp.int32),
            pltpu.VMEM((ECH, d), jnp.float32),
            pltpu.SemaphoreType.DMA,
        ],
    )
    return fn(y, src, dst, zeros_d)


def _sc_decode(z, ia, ib):
    n, d = z.shape

    def body(z_hbm, ia_hbm, ib_hbm, out_hbm, ia_v, ib_v, za_v, zb_v,
             out_v, sem):
        c = lax.axis_index("c")
        s = lax.axis_index("s")
        wid = c * NS + s
        pltpu.sync_copy(ia_hbm.at[pl.ds(wid * DPW, DPW)], ia_v)
        pltpu.sync_copy(ib_hbm.at[pl.ds(wid * DPW, DPW)], ib_v)

        def chunk(j, carry):
            pltpu.async_copy(z_hbm.at[ia_v.at[pl.ds(j * DCH, DCH)]],
                             za_v, sem).wait()
            pltpu.async_copy(z_hbm.at[ib_v.at[pl.ds(j * DCH, DCH)]],
                             zb_v, sem).wait()
            base = j * DCH
            for g in range(DCH // LANES):
                rows = g * LANES + lax.iota(jnp.int32, LANES)
                acc = jnp.zeros((LANES,), jnp.float32)
                for col in range(d):
                    cols = jnp.full((LANES,), col, jnp.int32)
                    a = plsc.load_gather(za_v, [rows, cols])
                    b = plsc.load_gather(zb_v, [rows, cols])
                    acc = acc + a * b
                out_v[pl.ds(base + g * LANES, LANES)] = acc
            return carry

        lax.fori_loop(0, DNCH, chunk, 0)
        pltpu.sync_copy(out_v, out_hbm.at[pl.ds(wid * DPW, DPW)])

    fn = pl.kernel(
        body,
        out_type=jax.ShapeDtypeStruct((ELP,), jnp.float32),
        mesh=_mesh(),
        compiler_params=pltpu.CompilerParams(use_tc_tiling_on_sc=False, needs_layout_passes=False),
        scratch_types=[
            pltpu.VMEM((DPW,), jnp.int32),
            pltpu.VMEM((DPW,), jnp.int32),
            pltpu.VMEM((DCH, d), jnp.float32),
            pltpu.VMEM((DCH, d), jnp.float32),
            pltpu.VMEM((DPW,), jnp.float32),
            pltpu.SemaphoreType.DMA,
        ],
    )
    return fn(z, ia, ib)


@jax.jit
def kernel(x, edge_index, edge_label_index, W_l1, b_l1, W_r1, W_l2, b_l2, W_r2):
    n, d_in = x.shape
    d_hid = W_l1.shape[1]
    d_out = W_l2.shape[1]
    e = edge_index.shape[1]
    el = edge_label_index.shape[1]

    epad = EP - e
    src = jnp.concatenate(
        [edge_index[0], jnp.zeros((epad,), jnp.int32)]).reshape(NW, ENCH, ECH)
    dst = jnp.concatenate(
        [edge_index[1], jnp.full((epad,), n, jnp.int32)]).reshape(NW, ENCH, ECH)
    zeros_hid = jnp.zeros((NPAD, d_hid), jnp.float32)
    zeros_out = jnp.zeros((NPAD, d_out), jnp.float32)
    zeros1 = jnp.zeros((NPAD,), jnp.float32)
    ones1 = jnp.ones((ECH,), jnp.float32)

    y1, r1 = _tc1(x, W_l1, W_r1, b_l1)
    agg1, deg = _sc_aggregate_deg(y1, src, dst, zeros_hid, zeros1, ones1)
    deg2d = deg.reshape(NC * NPAD, 1)
    r1p = jnp.concatenate([r1, jnp.zeros((NPAD - n, d_hid), jnp.float32)])
    y2, r2 = _tc2(agg1, deg2d, r1p, W_l2, W_r2, b_l2)
    agg2 = _sc_aggregate(y2[:n], src, dst, zeros_out)
    z = _tc3(agg2, deg2d, r2)

    pad = ELP - el
    ia = jnp.concatenate([edge_label_index[0], jnp.zeros((pad,), jnp.int32)])
    ib = jnp.concatenate([edge_label_index[1], jnp.zeros((pad,), jnp.int32)])
    out = _sc_decode(z[:n], ia, ib)
    return out[:el]


# speedup vs baseline: 3.5100x; 1.0241x over previous
import jax
import jax.numpy as jnp
from jax import lax
from jax.experimental import pallas as pl
from jax.experimental.pallas import tpu as pltpu
from jax.experimental.pallas import tpu_sc as plsc

NC = 2
NS = 16
NW = NC * NS
LANES = 16

NPAD = 10240
STRIPE = NPAD // NS
ECH = 64
ENCH = 160
EPW = ENCH * ECH
EP = NW * EPW
DCH = 128
DNCH = 26
DPW = DNCH * DCH
ELP = NW * DPW

_SC_PARAMS = dict(
    compiler_params=pltpu.CompilerParams(
        use_tc_tiling_on_sc=False, needs_layout_passes=False),
)


def _mesh():
    return plsc.VectorSubcoreMesh(core_axis_name="c", subcore_axis_name="s")


def _tc1_body(x_ref, wl_ref, wr_ref, b_ref, y_ref, r_ref):
    xv = x_ref[...]
    y_ref[...] = jnp.dot(xv, wl_ref[...], preferred_element_type=jnp.float32)
    r_ref[...] = (
        jnp.dot(xv, wr_ref[...], preferred_element_type=jnp.float32) + b_ref[...]
    )


def _tc1(x, W_l, W_r, b, br=2000):
    n, d = x.shape
    dh = W_l.shape[1]
    grid = n // br
    return pl.pallas_call(
        _tc1_body,
        grid=(grid,),
        in_specs=[
            pl.BlockSpec((br, d), lambda i: (i, 0)),
            pl.BlockSpec((d, dh), lambda i: (0, 0)),
            pl.BlockSpec((d, dh), lambda i: (0, 0)),
            pl.BlockSpec((1, dh), lambda i: (0, 0)),
        ],
        out_specs=[
            pl.BlockSpec((br, dh), lambda i: (i, 0)),
            pl.BlockSpec((br, dh), lambda i: (i, 0)),
        ],
        out_shape=[
            jax.ShapeDtypeStruct((n, dh), jnp.float32),
            jax.ShapeDtypeStruct((n, dh), jnp.float32),
        ],
    )(x, W_l, W_r, b.reshape(1, dh))


def _tc2_body(aggA, aggB, dgA, dgB, r1_ref, wl_ref, wr_ref, b_ref, y2_ref, r2_ref):
    deg = dgA[...] + dgB[...]
    inv = 1.0 / jnp.maximum(deg, 1.0)
    h = jnp.maximum((aggA[...] + aggB[...]) * inv + r1_ref[...], 0.0)
    y2_ref[...] = jnp.dot(h, wl_ref[...], preferred_element_type=jnp.float32)
    r2_ref[...] = (
        jnp.dot(h, wr_ref[...], preferred_element_type=jnp.float32) + b_ref[...]
    )


def _tc2(agg, deg2d, r1, W_l2, W_r2, b_l2, br=2048):
    dh = r1.shape[1]
    do = W_l2.shape[1]
    grid = NPAD // br
    off = NPAD // br
    return pl.pallas_call(
        _tc2_body,
        grid=(grid,),
        in_specs=[
            pl.BlockSpec((br, dh), lambda i: (i, 0)),
            pl.BlockSpec((br, dh), lambda i: (i + off, 0)),
            pl.BlockSpec((br, 1), lambda i: (i, 0)),
            pl.BlockSpec((br, 1), lambda i: (i + off, 0)),
            pl.BlockSpec((br, dh), lambda i: (i, 0)),
            pl.BlockSpec((dh, do), lambda i: (0, 0)),
            pl.BlockSpec((dh, do), lambda i: (0, 0)),
            pl.BlockSpec((1, do), lambda i: (0, 0)),
        ],
        out_specs=[
            pl.BlockSpec((br, do), lambda i: (i, 0)),
            pl.BlockSpec((br, do), lambda i: (i, 0)),
        ],
        out_shape=[
            jax.ShapeDtypeStruct((NPAD, do), jnp.float32),
            jax.ShapeDtypeStruct((NPAD, do), jnp.float32),
        ],
    )(agg, agg, deg2d, deg2d, r1, W_l2, W_r2, b_l2.reshape(1, do))


def _tc3_body(aggA, aggB, dgA, dgB, r2_ref, z_ref):
    deg = dgA[...] + dgB[...]
    inv = 1.0 / jnp.maximum(deg, 1.0)
    z_ref[...] = (aggA[...] + aggB[...]) * inv + r2_ref[...]


def _tc3(agg, deg2d, r2, br=2048):
    do = r2.shape[1]
    grid = NPAD // br
    off = NPAD // br
    return pl.pallas_call(
        _tc3_body,
        grid=(grid,),
        in_specs=[
            pl.BlockSpec((br, do), lambda i: (i, 0)),
            pl.BlockSpec((br, do), lambda i: (i + off, 0)),
            pl.BlockSpec((br, 1), lambda i: (i, 0)),
            pl.BlockSpec((br, 1), lambda i: (i + off, 0)),
            pl.BlockSpec((br, do), lambda i: (i, 0)),
        ],
        out_specs=pl.BlockSpec((br, do), lambda i: (i, 0)),
        out_shape=jax.ShapeDtypeStruct((NPAD, do), jnp.float32),
    )(agg, agg, deg2d, deg2d, r2)


def _agg_pipeline(y_hbm, src_hbm, dst_hbm, z_hbm, agg_out, agg_sh,
                  src_v, dst_v, rows0, rows1, gs0, gs1, deg_io):
    c = lax.axis_index("c")
    s = lax.axis_index("s")
    wid = c * NS + s
    row0 = s * STRIPE
    pltpu.sync_copy(src_hbm.at[wid], src_v)
    pltpu.sync_copy(dst_hbm.at[wid], dst_v)
    pltpu.async_copy(y_hbm.at[src_v.at[0]], rows0, gs0)
    pltpu.sync_copy(z_hbm.at[pl.ds(row0, STRIPE)],
                    agg_sh.at[pl.ds(row0, STRIPE)])
    if deg_io is not None:
        z1_hbm, ones_hbm, deg_out, deg_sh, ones_v = deg_io
        pltpu.sync_copy(z1_hbm.at[pl.ds(row0, STRIPE)],
                        deg_sh.at[pl.ds(row0, STRIPE)])
        pltpu.sync_copy(ones_hbm, ones_v)
    plsc.subcore_barrier()

    def pair(k, carry):
        j0 = 2 * k
        pltpu.make_async_copy(y_hbm.at[src_v.at[j0]], rows0, gs0).wait()
        pltpu.async_copy(y_hbm.at[src_v.at[j0 + 1]], rows1, gs1)
        pltpu.sync_copy(rows0, agg_sh.at[dst_v.at[j0]], add=True)
        if deg_io is not None:
            pltpu.sync_copy(ones_v, deg_sh.at[dst_v.at[j0]], add=True)
        pltpu.make_async_copy(y_hbm.at[src_v.at[j0 + 1]], rows1, gs1).wait()
        jn = jnp.minimum(j0 + 2, ENCH - 1)
        pltpu.async_copy(y_hbm.at[src_v.at[jn]], rows0, gs0)
        pltpu.sync_copy(rows1, agg_sh.at[dst_v.at[j0 + 1]], add=True)
        if deg_io is not None:
            pltpu.sync_copy(ones_v, deg_sh.at[dst_v.at[j0 + 1]], add=True)
        return carry

    lax.fori_loop(0, ENCH // 2, pair, 0)
    pltpu.make_async_copy(y_hbm.at[src_v.at[ENCH - 1]], rows0, gs0).wait()
    plsc.subcore_barrier()
    pltpu.sync_copy(agg_sh.at[pl.ds(row0, STRIPE)],
                    agg_out.at[pl.ds(c * NPAD + row0, STRIPE)])
    if deg_io is not None:
        z1_hbm, ones_hbm, deg_out, deg_sh, ones_v = deg_io
        pltpu.sync_copy(deg_sh.at[pl.ds(row0, STRIPE)],
                        deg_out.at[pl.ds(c * NPAD + row0, STRIPE)])


def _sc_aggregate_deg(y, src, dst, zeros_d, zeros1, ones1):
    n, d = y.shape

    def body(y_hbm, src_hbm, dst_hbm, z_hbm, z1_hbm, ones_hbm,
             agg_out, deg_out, agg_sh, deg_sh, src_v, dst_v,
             rows0, rows1, ones_v, gs0, gs1):
        _agg_pipeline(y_hbm, src_hbm, dst_hbm, z_hbm, agg_out, agg_sh,
                      src_v, dst_v, rows0, rows1, gs0, gs1,
                      (z1_hbm, ones_hbm, deg_out, deg_sh, ones_v))

    fn = pl.kernel(
        body,
        out_type=[
            jax.ShapeDtypeStruct((NC * NPAD, d), jnp.float32),
            jax.ShapeDtypeStruct((NC * NPAD,), jnp.float32),
        ],
        mesh=_mesh(),
        scratch_types=[
            pltpu.VMEM_SHARED((NPAD, d), jnp.float32),
            pltpu.VMEM_SHARED((NPAD,), jnp.float32),
            pltpu.VMEM((ENCH, ECH), jnp.int32),
            pltpu.VMEM((ENCH, ECH), jnp.int32),
            pltpu.VMEM((ECH, d), jnp.float32),
            pltpu.VMEM((ECH, d), jnp.float32),
            pltpu.VMEM((ECH,), jnp.float32),
            pltpu.SemaphoreType.DMA,
            pltpu.SemaphoreType.DMA,
        ],
        **_SC_PARAMS,
    )
    return fn(y, src, dst, zeros_d, zeros1, ones1)


def _sc_aggregate(y, src, dst, zeros_d):
    n, d = y.shape

    def body(y_hbm, src_hbm, dst_hbm, z_hbm,
             agg_out, agg_sh, src_v, dst_v, rows0, rows1, gs0, gs1):
        _agg_pipeline(y_hbm, src_hbm, dst_hbm, z_hbm, agg_out, agg_sh,
                      src_v, dst_v, rows0, rows1, gs0, gs1, None)

    fn = pl.kernel(
        body,
        out_type=jax.ShapeDtypeStruct((NC * NPAD, d), jnp.float32),
        mesh=_mesh(),
        scratch_types=[
            pltpu.VMEM_SHARED((NPAD, d), jnp.float32),
            pltpu.VMEM((ENCH, ECH), jnp.int32),
            pltpu.VMEM((ENCH, ECH), jnp.int32),
            pltpu.VMEM((ECH, d), jnp.float32),
            pltpu.VMEM((ECH, d), jnp.float32),
            pltpu.SemaphoreType.DMA,
            pltpu.SemaphoreType.DMA,
        ],
        **_SC_PARAMS,
    )
    return fn(y, src, dst, zeros_d)


def _sc_decode(z, ia, ib):
    n, d = z.shape

    def body(z_hbm, ia_hbm, ib_hbm, out_hbm, ia_v, ib_v,
             za0, zb0, za1, zb1, out_v, gs0, gs1):
        c = lax.axis_index("c")
        s = lax.axis_index("s")
        wid = c * NS + s
        pltpu.sync_copy(ia_hbm.at[pl.ds(wid * DPW, DPW)], ia_v)
        pltpu.sync_copy(ib_hbm.at[pl.ds(wid * DPW, DPW)], ib_v)
        pltpu.async_copy(z_hbm.at[ia_v.at[pl.ds(0, DCH)]], za0, gs0)
        pltpu.async_copy(z_hbm.at[ib_v.at[pl.ds(0, DCH)]], zb0, gs0)

        def compute(j, za_v, zb_v):
            base = j * DCH
            for g in range(DCH // LANES):
                rows = g * LANES + lax.iota(jnp.int32, LANES)
                acc = jnp.zeros((LANES,), jnp.float32)
                for col in range(d):
                    cols = jnp.full((LANES,), col, jnp.int32)
                    a = plsc.load_gather(za_v, [rows, cols])
                    b = plsc.load_gather(zb_v, [rows, cols])
                    acc = acc + a * b
                out_v[pl.ds(base + g * LANES, LANES)] = acc

        def pair(k, carry):
            j0 = 2 * k
            pltpu.make_async_copy(z_hbm.at[ia_v.at[pl.ds(0, DCH)]], za0,
                                  gs0).wait()
            pltpu.make_async_copy(z_hbm.at[ib_v.at[pl.ds(0, DCH)]], zb0,
                                  gs0).wait()
            o1 = (j0 + 1) * DCH
            pltpu.async_copy(z_hbm.at[ia_v.at[pl.ds(o1, DCH)]], za1, gs1)
            pltpu.async_copy(z_hbm.at[ib_v.at[pl.ds(o1, DCH)]], zb1, gs1)
            compute(j0, za0, zb0)
            pltpu.make_async_copy(z_hbm.at[ia_v.at[pl.ds(o1, DCH)]], za1,
                                  gs1).wait()
            pltpu.make_async_copy(z_hbm.at[ib_v.at[pl.ds(o1, DCH)]], zb1,
                                  gs1).wait()
            on = jnp.minimum(j0 + 2, DNCH - 1) * DCH
            pltpu.async_copy(z_hbm.at[ia_v.at[pl.ds(on, DCH)]], za0, gs0)
            pltpu.async_copy(z_hbm.at[ib_v.at[pl.ds(on, DCH)]], zb0, gs0)
            compute(j0 + 1, za1, zb1)
            return carry

        lax.fori_loop(0, DNCH // 2, pair, 0)
        pltpu.make_async_copy(z_hbm.at[ia_v.at[pl.ds(0, DCH)]], za0,
                              gs0).wait()
        pltpu.make_async_copy(z_hbm.at[ib_v.at[pl.ds(0, DCH)]], zb0,
                              gs0).wait()
        pltpu.sync_copy(out_v, out_hbm.at[pl.ds(wid * DPW, DPW)])

    fn = pl.kernel(
        body,
        out_type=jax.ShapeDtypeStruct((ELP,), jnp.float32),
        mesh=_mesh(),
        scratch_types=[
            pltpu.VMEM((DPW,), jnp.int32),
            pltpu.VMEM((DPW,), jnp.int32),
            pltpu.VMEM((DCH, d), jnp.float32),
            pltpu.VMEM((DCH, d), jnp.float32),
            pltpu.VMEM((DCH, d), jnp.float32),
            pltpu.VMEM((DCH, d), jnp.float32),
            pltpu.VMEM((DPW,), jnp.float32),
            pltpu.SemaphoreType.DMA,
            pltpu.SemaphoreType.DMA,
        ],
        **_SC_PARAMS,
    )
    return fn(z, ia, ib)


@jax.jit
def kernel(x, edge_index, edge_label_index, W_l1, b_l1, W_r1, W_l2, b_l2, W_r2):
    n, d_in = x.shape
    d_hid = W_l1.shape[1]
    d_out = W_l2.shape[1]
    e = edge_index.shape[1]
    el = edge_label_index.shape[1]

    epad = EP - e
    src = jnp.concatenate(
        [edge_index[0], jnp.zeros((epad,), jnp.int32)]).reshape(NW, ENCH, ECH)
    dst = jnp.concatenate(
        [edge_index[1], jnp.full((epad,), n, jnp.int32)]).reshape(NW, ENCH, ECH)
    zeros_hid = jnp.zeros((NPAD, d_hid), jnp.float32)
    zeros_out = jnp.zeros((NPAD, d_out), jnp.float32)
    zeros1 = jnp.zeros((NPAD,), jnp.float32)
    ones1 = jnp.ones((ECH,), jnp.float32)

    y1, r1 = _tc1(x, W_l1, W_r1, b_l1)
    agg1, deg = _sc_aggregate_deg(y1, src, dst, zeros_hid, zeros1, ones1)
    deg2d = deg.reshape(NC * NPAD, 1)
    r1p = jnp.concatenate([r1, jnp.zeros((NPAD - n, d_hid), jnp.float32)])
    y2, r2 = _tc2(agg1, deg2d, r1p, W_l2, W_r2, b_l2)
    agg2 = _sc_aggregate(y2[:n], src, dst, zeros_out)
    z = _tc3(agg2, deg2d, r2)

    pad = ELP - el
    ia = jnp.concatenate([edge_label_index[0], jnp.zeros((pad,), jnp.int32)])
    ib = jnp.concatenate([edge_label_index[1], jnp.zeros((pad,), jnp.int32)])
    out = _sc_decode(z[:n], ia, ib)
    return out[:el]
